# baseline (speedup 1.0000x reference)
import functools

import jax
import jax.numpy as jnp
from jax import lax
from jax.experimental import pallas as pl
from jax.experimental.pallas import tpu as pltpu
from jax.experimental.pallas import tpu_sc as plsc

N = 10000
D = 128
NB = 3
NC, NS, LANES = 2, 16, 16
NW = NC * NS
NP = 10240
RPT = NP // NS
CHUNK = 128
ACH = 128
NIB = 16
BT = 2048


def _sc_mesh():
    return plsc.VectorSubcoreMesh(
        core_axis_name="c", subcore_axis_name="s",
        num_cores=NC, num_subcores=NS)


@functools.lru_cache(maxsize=None)
def _deg_call(nch):
    def body(dst_hbm, out_hbm, didx, ones_v, stage, acc_sh):
        c = lax.axis_index("c")
        s = lax.axis_index("s")
        w = s * NC + c
        pltpu.sync_copy(dst_hbm.at[w], didx)

        def fill_ones(i, _):
            ones_v[pl.ds(i * LANES, LANES)] = jnp.ones((LANES,), jnp.float32)
            return 0
        lax.fori_loop(0, CHUNK // LANES, fill_ones, 0)

        def fill_zero(i, _):
            stage[pl.ds(i * LANES, LANES)] = jnp.zeros((LANES,), jnp.float32)
            return 0
        lax.fori_loop(0, RPT // LANES, fill_zero, 0)
        pltpu.sync_copy(stage, acc_sh.at[pl.ds(s * RPT, RPT)])
        plsc.subcore_barrier()

        def add_chunk(j, _):
            pltpu.sync_copy(ones_v, acc_sh.at[didx.at[j]], add=True)
            return 0
        lax.fori_loop(0, nch, add_chunk, 0)
        plsc.subcore_barrier()

        pltpu.sync_copy(acc_sh.at[pl.ds(s * RPT, RPT)],
                        out_hbm.at[c, pl.ds(s * RPT, RPT)])

    return pl.kernel(
        body,
        out_type=jax.ShapeDtypeStruct((NC, NP), jnp.float32),
        mesh=_sc_mesh(),
        scratch_types=[
            pltpu.VMEM((nch, CHUNK), jnp.int32),
            pltpu.VMEM((CHUNK,), jnp.float32),
            pltpu.VMEM((RPT,), jnp.float32),
            pltpu.VMEM_SHARED((NP,), jnp.float32),
        ],
    )


@functools.lru_cache(maxsize=None)
def _agg_call(nblk):
    SS = 8
    assert NIB % SS == 0 and NIB // SS >= 2

    def body(u0_hbm, u1_hbm, u2_hbm, src_hbm, dst_hbm, out_hbm,
             sidx, didx, b0, b1, g0, g1, s0, s1, acc_sh):
        c = lax.axis_index("c")
        s = lax.axis_index("s")
        w = s * NC + c
        buf = [b0, b1]
        gsem = [g0, g1]
        ssem = [s0, s1]

        for q, u_hbm in enumerate((u0_hbm, u1_hbm, u2_hbm)):
            def gather(k, r):
                pltpu.async_copy(u_hbm.at[sidx.at[k]], buf[r], gsem[r])

            def wait_gather(k, r):
                pltpu.make_async_copy(u_hbm.at[sidx.at[k]], buf[r],
                                      gsem[r]).wait()

            def scat(k, r):
                pltpu.async_copy(buf[r], acc_sh.at[didx.at[k]],
                                 ssem[r], add=True)

            def wait_scat(k, r):
                pltpu.make_async_copy(buf[r], acc_sh.at[didx.at[k]],
                                      ssem[r]).wait()

            def zb(i, _):
                r = i // (D // LANES)
                k = i % (D // LANES)
                b0[r, pl.ds(k * LANES, LANES)] = jnp.zeros((LANES,),
                                                           jnp.float32)
                return 0
            lax.fori_loop(0, ACH * (D // LANES), zb, 0)
            for t in range(RPT // ACH):
                pltpu.sync_copy(b0, acc_sh.at[pl.ds(s * RPT + t * ACH, ACH)])
            plsc.subcore_barrier()

            for b in range(nblk):
                pltpu.sync_copy(src_hbm.at[w, b], sidx)
                pltpu.sync_copy(dst_hbm.at[w, b], didx)
                gather(0, 0)
                for k in range(SS):
                    if k >= 1:
                        wait_scat(k - 1, (k - 1) % 2)
                    if k + 1 < NIB:
                        gather(k + 1, (k + 1) % 2)
                    wait_gather(k, k % 2)
                    scat(k, k % 2)

                def sstep(t, _):
                    base = t * SS
                    for u in range(SS):
                        k = base + u
                        wait_scat(k - 1, (u + 1) % 2)
                        gather(k + 1, (u + 1) % 2)
                        wait_gather(k, u % 2)
                        scat(k, u % 2)
                    return 0
                lax.fori_loop(1, NIB // SS - 1, sstep, 0)

                for k in range(NIB - SS, NIB):
                    wait_scat(k - 1, (k - 1) % 2)
                    if k + 1 < NIB:
                        gather(k + 1, (k + 1) % 2)
                    wait_gather(k, k % 2)
                    scat(k, k % 2)
                wait_scat(NIB - 1, (NIB - 1) % 2)

            plsc.subcore_barrier()
            pltpu.sync_copy(acc_sh.at[pl.ds(s * RPT, RPT)],
                            out_hbm.at[c, q, pl.ds(s * RPT, RPT)])

    return pl.kernel(
        body,
        out_type=jax.ShapeDtypeStruct((NC, NB, NP, D), jnp.float32),
        mesh=_sc_mesh(),
        scratch_types=[
            pltpu.VMEM((NIB, ACH), jnp.int32),
            pltpu.VMEM((NIB, ACH), jnp.int32),
            pltpu.VMEM((ACH, D), jnp.float32),
            pltpu.VMEM((ACH, D), jnp.float32),
            pltpu.SemaphoreType.DMA,
            pltpu.SemaphoreType.DMA,
            pltpu.SemaphoreType.DMA,
            pltpu.SemaphoreType.DMA,
            pltpu.VMEM_SHARED((NP, D), jnp.float32),
        ],
    )


def _dinv_of(degt_ref):
    return lax.rsqrt(degt_ref[:, 0:1] + degt_ref[:, 1:2] + 1.0)


def _mm(a, b):
    return jnp.dot(a, b, preferred_element_type=jnp.float32)


def _head_body(degt_ref, x_ref, wa_ref, wb_ref, wc_ref,
               u0_ref, u1_ref, u2_ref):
    dinv = _dinv_of(degt_ref)
    x = x_ref[...]
    u0_ref[...] = _mm(x, wa_ref[...]) * dinv
    u1_ref[...] = _mm(x, wb_ref[...]) * dinv
    u2_ref[...] = _mm(x, wc_ref[...]) * dinv


_head = pl.pallas_call(
    _head_body,
    grid=(NP // BT,),
    in_specs=[
        pl.BlockSpec((BT, NC), lambda i: (i, 0)),
        pl.BlockSpec((BT, D), lambda i: (i, 0)),
        pl.BlockSpec((D, D), lambda i: (0, 0)),
        pl.BlockSpec((D, D), lambda i: (0, 0)),
        pl.BlockSpec((D, D), lambda i: (0, 0)),
    ],
    out_specs=[pl.BlockSpec((BT, D), lambda i: (i, 0))] * NB,
    out_shape=[jax.ShapeDtypeStruct((NP, D), jnp.float32)] * NB,
)


def _tail_body(p_ref, u0_ref, u1_ref, u2_ref, degt_ref, ba_ref, bb_ref,
               bc_ref, wl_ref, bl_ref, wn_refs, o_refs, *, final):
    i = pl.program_id(0)
    dinv = _dinv_of(degt_ref)
    hsum = jnp.zeros((BT, D), jnp.float32)
    for bi, (u_ref, b_ref) in enumerate(((u0_ref, ba_ref), (u1_ref, bb_ref),
                                         (u2_ref, bc_ref))):
        t = p_ref[0, bi] + p_ref[1, bi] + u_ref[...]
        hb = jnp.maximum(t * dinv + b_ref[...], 0.0)
        hsum = hsum + _mm(hb, wl_ref[bi * D:(bi + 1) * D])
    h = jnp.maximum(hsum + bl_ref[...], 0.0)
    if final:
        wf_ref, bf_ref = wn_refs
        o_refs[0][...] = _mm(h, wf_ref[...]) + bf_ref[...]
    else:
        rows = i * BT + lax.broadcasted_iota(jnp.int32, (BT, 1), 0)
        h = jnp.where(rows < N, h, 0.0)
        for w_ref, o_ref in zip(wn_refs, o_refs):
            o_ref[...] = _mm(h, w_ref[...]) * dinv


def _mid_body(p_ref, u0_ref, u1_ref, u2_ref, degt_ref, ba_ref, bb_ref,
              bc_ref, wl_ref, bl_ref, wa_ref, wb_ref, wc_ref,
              o0_ref, o1_ref, o2_ref):
    _tail_body(p_ref, u0_ref, u1_ref, u2_ref, degt_ref, ba_ref, bb_ref,
               bc_ref, wl_ref, bl_ref, (wa_ref, wb_ref, wc_ref),
               (o0_ref, o1_ref, o2_ref), final=False)


def _fin_body(p_ref, u0_ref, u1_ref, u2_ref, degt_ref, ba_ref, bb_ref,
              bc_ref, wl_ref, bl_ref, wf_ref, bf_ref, o_ref):
    _tail_body(p_ref, u0_ref, u1_ref, u2_ref, degt_ref, ba_ref, bb_ref,
               bc_ref, wl_ref, bl_ref, (wf_ref, bf_ref), (o_ref,),
               final=True)


_COMMON_SPECS = [
    pl.BlockSpec((NC, NB, BT, D), lambda i: (0, 0, i, 0)),
    pl.BlockSpec((BT, D), lambda i: (i, 0)),
    pl.BlockSpec((BT, D), lambda i: (i, 0)),
    pl.BlockSpec((BT, D), lambda i: (i, 0)),
    pl.BlockSpec((BT, NC), lambda i: (i, 0)),
    pl.BlockSpec((1, D), lambda i: (0, 0)),
    pl.BlockSpec((1, D), lambda i: (0, 0)),
    pl.BlockSpec((1, D), lambda i: (0, 0)),
    pl.BlockSpec((3 * D, D), lambda i: (0, 0)),
    pl.BlockSpec((1, D), lambda i: (0, 0)),
]

_mid = pl.pallas_call(
    _mid_body,
    grid=(NP // BT,),
    in_specs=_COMMON_SPECS + [pl.BlockSpec((D, D), lambda i: (0, 0))] * NB,
    out_specs=[pl.BlockSpec((BT, D), lambda i: (i, 0))] * NB,
    out_shape=[jax.ShapeDtypeStruct((NP, D), jnp.float32)] * NB,
)

_fin = pl.pallas_call(
    _fin_body,
    grid=(NP // BT,),
    in_specs=_COMMON_SPECS + [
        pl.BlockSpec((D, D), lambda i: (0, 0)),
        pl.BlockSpec((1, D), lambda i: (0, 0)),
    ],
    out_specs=pl.BlockSpec((BT, D), lambda i: (i, 0)),
    out_shape=jax.ShapeDtypeStruct((NP, D), jnp.float32),
)


def kernel(x, edge_index, W1a, b1a, W1b, b1b, W1c, b1c, Wl1, bl1,
           W2a, b2a, W2b, b2b, W2c, b2c, Wl2, bl2, Wfc, bfc):
    E = edge_index.shape[1]
    nblk = -(-E // (NW * NIB * ACH))
    pad = NW * nblk * NIB * ACH - E
    nchd = nblk * NIB * ACH // CHUNK
    idx = edge_index.astype(jnp.int32)
    fill = jnp.full((pad,), N, jnp.int32)
    src_flat = jnp.concatenate([idx[0], fill])
    dst_flat = jnp.concatenate([idx[1], fill])
    src = src_flat.reshape(NW, nblk, NIB, ACH)
    dst = dst_flat.reshape(NW, nblk, NIB, ACH)
    dst_deg = dst_flat.reshape(NW, nchd, CHUNK)
    xpad = jnp.pad(x, ((0, NP - N), (0, 0)))

    deg_parts = _deg_call(nchd)(dst_deg)
    degt = deg_parts.T

    u1a, u1b, u1c = _head(degt, xpad, W1a, W1b, W1c)
    p1 = _agg_call(nblk)(u1a, u1b, u1c, src, dst)
    u2a, u2b, u2c = _mid(p1, u1a, u1b, u1c, degt,
                         b1a.reshape(1, D), b1b.reshape(1, D),
                         b1c.reshape(1, D), Wl1, bl1.reshape(1, D),
                         W2a, W2b, W2c)
    p2 = _agg_call(nblk)(u2a, u2b, u2c, src, dst)
    wf = jnp.pad(Wfc, ((0, 0), (0, D - Wfc.shape[1])))
    bf = jnp.broadcast_to(bfc[None, :], (1, D))
    out = _fin(p2, u2a, u2b, u2c, degt,
               b2a.reshape(1, D), b2b.reshape(1, D), b2c.reshape(1, D),
               Wl2, bl2.reshape(1, D), wf, bf)
    return out[:N, :1]
